# baseline (speedup 1.0000x reference)
import functools

import jax
import jax.numpy as jnp
from jax import lax
from jax.experimental import pallas as pl
from jax.experimental.pallas import tpu as pltpu
from jax.experimental.pallas import tpu_sc as plsc

N = 50000
C = 128
NCOEF = 25
NW = 32
S = 1584
R = 48
CH = S // R
NB = 3
GROUPS = CH // NB
LA = 1

BT = 400

_mesh = plsc.VectorSubcoreMesh(core_axis_name="c", subcore_axis_name="s")


@functools.partial(
    pl.kernel,
    mesh=_mesh,
    out_type=jax.ShapeDtypeStruct((N, C), jnp.float32),
    scratch_types=[
        pltpu.VMEM((S,), jnp.int32),
        pltpu.VMEM((NB, R, C), jnp.float32),
        pltpu.VMEM_SHARED((100, C), jnp.float32),
        pltpu.SemaphoreType.DMA,
        pltpu.SemaphoreType.DMA,
        pltpu.SemaphoreType.DMA,
        pltpu.SemaphoreType.DMA,
        pltpu.SemaphoreType.DMA,
        pltpu.SemaphoreType.DMA,
    ],
)
def _gather_sc(idx_hbm, table_hbm, out_hbm, idx_v, buf, table_s,
               gsem0, gsem1, gsem2, wsem0, wsem1, wsem2):
    cid = lax.axis_index("c")
    sid = lax.axis_index("s")
    wid = sid * 2 + cid
    gsem = (gsem0, gsem1, gsem2)
    wsem = (wsem0, wsem1, wsem2)
    base_w = jnp.minimum(wid * S, N - S)

    pltpu.sync_copy(idx_hbm.at[pl.ds(base_w, S)], idx_v)

    @pl.when(sid == 0)
    def _():
        pltpu.sync_copy(table_hbm, table_s)

    plsc.subcore_barrier()

    for b in range(LA):
        pltpu.async_copy(
            table_s.at[idx_v.at[pl.ds(b * R, R)]], buf.at[b], gsem[b]
        )

    def group(g, carry):
        for b in range(NB):
            c = NB * g + b
            pltpu.make_async_copy(
                table_s.at[idx_v.at[pl.ds(0, R)]], buf.at[b], gsem[b]
            ).wait()
            pltpu.async_copy(
                buf.at[b], out_hbm.at[pl.ds(base_w + c * R, R)], wsem[b]
            )

            bn = (b + LA) % NB

            @pl.when(jnp.logical_and(c >= NB - LA, c + LA < CH))
            def _():
                pltpu.make_async_copy(
                    buf.at[bn], out_hbm.at[pl.ds(0, R)], wsem[bn]
                ).wait()

            @pl.when(c + LA < CH)
            def _():
                pltpu.async_copy(
                    table_s.at[idx_v.at[pl.ds((c + LA) * R, R)]],
                    buf.at[bn], gsem[bn],
                )
        return carry

    lax.fori_loop(0, GROUPS, group, 0)

    for b in range(NB):
        pltpu.make_async_copy(
            buf.at[b], out_hbm.at[pl.ds(0, R)], wsem[b]
        ).wait()


def _expand_tc(g_ref, o_ref):
    o_ref[:, 0:C] = g_ref[...]
    o_ref[:, C:] = jnp.zeros((BT, (NCOEF - 1) * C), jnp.float32)


def kernel(atomic_numbers, embedding_table):
    idx = atomic_numbers.astype(jnp.int32)
    g = _gather_sc(idx, embedding_table)
    out2d = pl.pallas_call(
        _expand_tc,
        grid=(N // BT,),
        in_specs=[pl.BlockSpec((BT, C), lambda i: (i, 0))],
        out_specs=pl.BlockSpec((BT, NCOEF * C), lambda i: (i, 0)),
        out_shape=jax.ShapeDtypeStruct((N, NCOEF * C), jnp.float32),
    )(g)
    return out2d.reshape(N, NCOEF, C)


# speedup vs baseline: 1.0001x; 1.0001x over previous
import functools

import jax
import jax.numpy as jnp
from jax import lax
from jax.experimental import pallas as pl
from jax.experimental.pallas import tpu as pltpu
from jax.experimental.pallas import tpu_sc as plsc

N = 50000
C = 128
NCOEF = 25
NW = 32
S = 1584
R = 48
CH = S // R
NB = 3
GROUPS = CH // NB
LA = 1

BT = 400

_mesh = plsc.VectorSubcoreMesh(core_axis_name="c", subcore_axis_name="s")


@functools.partial(
    pl.kernel,
    mesh=_mesh,
    out_type=jax.ShapeDtypeStruct((N, 1, C), jnp.float32),
    scratch_types=[
        pltpu.VMEM((S,), jnp.int32),
        pltpu.VMEM((NB, R, 1, C), jnp.float32),
        pltpu.VMEM_SHARED((100, 1, C), jnp.float32),
        pltpu.SemaphoreType.DMA,
        pltpu.SemaphoreType.DMA,
        pltpu.SemaphoreType.DMA,
        pltpu.SemaphoreType.DMA,
        pltpu.SemaphoreType.DMA,
        pltpu.SemaphoreType.DMA,
    ],
)
def _gather_sc(idx_hbm, table_hbm, out_hbm, idx_v, buf, table_s,
               gsem0, gsem1, gsem2, wsem0, wsem1, wsem2):
    cid = lax.axis_index("c")
    sid = lax.axis_index("s")
    wid = sid * 2 + cid
    gsem = (gsem0, gsem1, gsem2)
    wsem = (wsem0, wsem1, wsem2)
    base_w = jnp.minimum(wid * S, N - S)

    pltpu.sync_copy(idx_hbm.at[pl.ds(base_w, S)], idx_v)

    @pl.when(sid == 0)
    def _():
        pltpu.sync_copy(table_hbm, table_s)

    plsc.subcore_barrier()

    for b in range(LA):
        pltpu.async_copy(
            table_s.at[idx_v.at[pl.ds(b * R, R)]], buf.at[b], gsem[b]
        )

    def group(g, carry):
        for b in range(NB):
            c = NB * g + b
            pltpu.make_async_copy(
                table_s.at[idx_v.at[pl.ds(0, R)]], buf.at[b], gsem[b]
            ).wait()
            pltpu.async_copy(
                buf.at[b], out_hbm.at[pl.ds(base_w + c * R, R)], wsem[b]
            )

            bn = (b + LA) % NB

            @pl.when(jnp.logical_and(c >= NB - LA, c + LA < CH))
            def _():
                pltpu.make_async_copy(
                    buf.at[bn], out_hbm.at[pl.ds(0, R)], wsem[bn]
                ).wait()

            @pl.when(c + LA < CH)
            def _():
                pltpu.async_copy(
                    table_s.at[idx_v.at[pl.ds((c + LA) * R, R)]],
                    buf.at[bn], gsem[bn],
                )
        return carry

    lax.fori_loop(0, GROUPS, group, 0)

    for b in range(NB):
        pltpu.make_async_copy(
            buf.at[b], out_hbm.at[pl.ds(0, R)], wsem[b]
        ).wait()


def _expand_tc(g_ref, o_ref):
    o_ref[:, 0:C] = g_ref[:, 0, :]
    o_ref[:, C:] = jnp.zeros((BT, (NCOEF - 1) * C), jnp.float32)


def kernel(atomic_numbers, embedding_table):
    idx = atomic_numbers.astype(jnp.int32)
    g = _gather_sc(idx, embedding_table.reshape(100, 1, C))
    out2d = pl.pallas_call(
        _expand_tc,
        grid=(N // BT,),
        in_specs=[pl.BlockSpec((BT, 1, C), lambda i: (i, 0, 0))],
        out_specs=pl.BlockSpec((BT, NCOEF * C), lambda i: (i, 0)),
        out_shape=jax.ShapeDtypeStruct((N, NCOEF * C), jnp.float32),
    )(g)
    return out2d.reshape(N, NCOEF, C)
